# Initial kernel scaffold; baseline (speedup 1.0000x reference)
import jax
import jax.numpy as jnp
from jax.experimental import pallas as pl


def kernel(x, edge_index, W1, b1, W2, b2, Wc, bc):
    raise NotImplementedError("write your pallas kernel here")


# speedup vs baseline: 9.7587x; 9.7587x over previous
import functools

import jax
import jax.numpy as jnp
from jax import lax
from jax.experimental import pallas as pl
from jax.experimental.pallas import tpu as pltpu
from jax.experimental.pallas import tpu_sc as plsc

N = 10000
E = 320000
D_IN = 128
H1 = 128
H2 = 64
NC_OUT = 10

NCORES = 2
NSUB = 16
NWORK = NCORES * NSUB
CHUNK = 128
CHUNKS_PW = 80
IB = 40
NBLK = CHUNKS_PW // IB
EPW = CHUNK * CHUNKS_PW
EPAD = EPW * NWORK
NPAD = 10240
ROWS_PT = NPAD // NSUB
RB_CHUNKS = ROWS_PT // CHUNK

_mesh = plsc.VectorSubcoreMesh(core_axis_name="c", subcore_axis_name="s")


@functools.partial(
    pl.kernel,
    out_type=jax.ShapeDtypeStruct((NCORES, NPAD), jnp.float32),
    mesh=_mesh,
    scratch_types=[
        pltpu.VMEM_SHARED((NPAD,), jnp.float32),
        pltpu.VMEM((CHUNKS_PW, CHUNK), jnp.int32),
        pltpu.VMEM((CHUNK,), jnp.float32),
        pltpu.VMEM((ROWS_PT,), jnp.float32),
    ],
)
def _deg_kernel(dst_hbm, deg_out, acc, dst_v, ones_v, buf_v):
    cc = lax.axis_index("c")
    ss = lax.axis_index("s")
    w = cc * NSUB + ss

    pltpu.sync_copy(dst_hbm.at[w], dst_v)
    for i in range(ROWS_PT // 16):
        buf_v[pl.ds(i * 16, 16)] = jnp.zeros((16,), jnp.float32)
    for i in range(CHUNK // 16):
        ones_v[pl.ds(i * 16, 16)] = jnp.ones((16,), jnp.float32)
    pltpu.sync_copy(buf_v, acc.at[pl.ds(ROWS_PT * ss, ROWS_PT)])
    plsc.subcore_barrier()

    @pl.loop(0, CHUNKS_PW)
    def _(i):
        pltpu.sync_copy(ones_v, acc.at[dst_v.at[i]], add=True)

    plsc.subcore_barrier()
    pltpu.sync_copy(acc.at[pl.ds(ROWS_PT * ss, ROWS_PT)], buf_v)
    pltpu.sync_copy(buf_v, deg_out.at[cc, pl.ds(ROWS_PT * ss, ROWS_PT)])


def _make_agg_kernel(width):
    @functools.partial(
        pl.kernel,
        out_type=jax.ShapeDtypeStruct((NCORES, NPAD, width), jnp.float32),
        mesh=_mesh,
        scratch_types=[
            pltpu.VMEM_SHARED((NPAD, width), jnp.float32),
            pltpu.VMEM((IB, CHUNK), jnp.int32),
            pltpu.VMEM((IB, CHUNK), jnp.int32),
            pltpu.VMEM((CHUNK, width), jnp.float32),
            pltpu.VMEM((CHUNK, width), jnp.float32),
            pltpu.SemaphoreType.DMA,
            pltpu.SemaphoreType.DMA,
        ],
    )
    def _agg(g_hbm, src_hbm, dst_hbm, zz_hbm, out, acc,
             src_v, dst_v, rows0, rows1, sem0, sem1):
        cc = lax.axis_index("c")
        ss = lax.axis_index("s")
        w = cc * NSUB + ss

        pltpu.sync_copy(zz_hbm, rows0)
        for k in range(RB_CHUNKS):
            pltpu.sync_copy(rows0, acc.at[pl.ds(ROWS_PT * ss + CHUNK * k, CHUNK)])
        plsc.subcore_barrier()

        for blk in range(NBLK):
            pltpu.sync_copy(src_hbm.at[w, pl.ds(IB * blk, IB)], src_v)
            pltpu.sync_copy(dst_hbm.at[w, pl.ds(IB * blk, IB)], dst_v)
            pltpu.async_copy(g_hbm.at[src_v.at[0]], rows0, sem0)
            pltpu.async_copy(g_hbm.at[src_v.at[1]], rows1, sem1)

            @pl.loop(0, IB // 2)
            def _(gi):
                c0 = 2 * gi
                c1 = 2 * gi + 1
                pltpu.make_async_copy(g_hbm.at[src_v.at[c0]], rows0, sem0).wait()
                pltpu.sync_copy(rows0, acc.at[dst_v.at[c0]], add=True)

                @pl.when(c0 + 2 < IB)
                def _():
                    pltpu.async_copy(g_hbm.at[src_v.at[c0 + 2]], rows0, sem0)

                pltpu.make_async_copy(g_hbm.at[src_v.at[c1]], rows1, sem1).wait()
                pltpu.sync_copy(rows1, acc.at[dst_v.at[c1]], add=True)

                @pl.when(c1 + 2 < IB)
                def _():
                    pltpu.async_copy(g_hbm.at[src_v.at[c1 + 2]], rows1, sem1)

        plsc.subcore_barrier()
        for k in range(RB_CHUNKS):
            base = ROWS_PT * ss + CHUNK * k
            pltpu.sync_copy(acc.at[pl.ds(base, CHUNK)], rows0)
            pltpu.sync_copy(rows0, out.at[cc, pl.ds(base, CHUNK)])

    return _agg


_agg128 = _make_agg_kernel(H1)
H2P = 128


_BLK = 1000
_GRID = N // _BLK


def _tc1_body(p0, p1, x, w1, g1_out, dinv_out):
    deg = p0[...] + p1[...] + 1.0
    dinv = lax.rsqrt(deg)
    h = jnp.dot(x[...], w1[...], preferred_element_type=jnp.float32)
    g1_out[...] = h * dinv
    dinv_out[...] = dinv


def _tc2_body(p, g1, dinv, b1, w2, g2_out):
    pv = p[...]
    s = (pv[0] + pv[1] + g1[...]) * dinv[...]
    h1 = jnp.maximum(s + b1[...], 0.0)
    h2 = jnp.dot(h1, w2[...], preferred_element_type=jnp.float32)
    g2_out[...] = h2 * dinv[...]


def _tc3_body(q, g2, dinv, b2, wc, bc, out):
    qv = q[...]
    s = (qv[0] + qv[1] + g2[...]) * dinv[...]
    h2 = jnp.maximum(s + b2[...], 0.0)
    out[...] = jnp.dot(h2, wc[...], preferred_element_type=jnp.float32) + bc[...]


def _row_spec(width):
    return pl.BlockSpec((_BLK, width), lambda i: (i, 0))


def _pair_spec(width):
    return pl.BlockSpec((NCORES, _BLK, width), lambda i: (0, i, 0))


def _full_spec(a, b):
    return pl.BlockSpec((a, b), lambda i: (0, 0))


_tc1 = pl.pallas_call(
    _tc1_body,
    grid=(_GRID,),
    in_specs=[_row_spec(1), _row_spec(1), _row_spec(D_IN), _full_spec(D_IN, H1)],
    out_specs=[_row_spec(H1), _row_spec(1)],
    out_shape=[
        jax.ShapeDtypeStruct((N, H1), jnp.float32),
        jax.ShapeDtypeStruct((N, 1), jnp.float32),
    ],
)

_tc2 = pl.pallas_call(
    _tc2_body,
    grid=(_GRID,),
    in_specs=[_pair_spec(H1), _row_spec(H1), _row_spec(1),
              _full_spec(1, H1), _full_spec(H1, H2P)],
    out_specs=[_row_spec(H2P)],
    out_shape=[jax.ShapeDtypeStruct((N, H2P), jnp.float32)],
)

_tc3 = pl.pallas_call(
    _tc3_body,
    grid=(_GRID,),
    in_specs=[_pair_spec(H2P), _row_spec(H2P), _row_spec(1),
              _full_spec(1, H2P), _full_spec(H2P, NC_OUT), _full_spec(1, NC_OUT)],
    out_specs=[_row_spec(NC_OUT)],
    out_shape=[jax.ShapeDtypeStruct((N, NC_OUT), jnp.float32)],
)


def kernel(x, edge_index, W1, b1, W2, b2, Wc, bc):
    pad = EPAD - E
    src = jnp.concatenate([edge_index[0], jnp.zeros((pad,), jnp.int32)])
    dst = jnp.concatenate([edge_index[1], jnp.full((pad,), NPAD - 1, jnp.int32)])
    src_r = src.reshape(NWORK, CHUNKS_PW, CHUNK)
    dst_r = dst.reshape(NWORK, CHUNKS_PW, CHUNK)
    zz = jnp.zeros((CHUNK, H1), jnp.float32)

    degp = _deg_kernel(dst_r)
    p0 = degp[0, :N].reshape(N, 1)
    p1 = degp[1, :N].reshape(N, 1)

    W2p = jnp.pad(W2, ((0, 0), (0, H2P - H2)))
    b2p = jnp.pad(b2, (0, H2P - H2)).reshape(1, H2P)
    Wcp = jnp.pad(Wc, ((0, H2P - H2), (0, 0)))

    g1, dinv = _tc1(p0, p1, x, W1)
    part1 = _agg128(g1, src_r, dst_r, zz)
    (g2,) = _tc2(part1, g1, dinv, b1.reshape(1, H1), W2p)
    part2 = _agg128(g2, src_r, dst_r, zz)
    (out,) = _tc3(part2, g2, dinv, b2p, Wcp, bc.reshape(1, NC_OUT))
    return out


# speedup vs baseline: 10.3745x; 1.0631x over previous
import functools

import jax
import jax.numpy as jnp
from jax import lax
from jax.experimental import pallas as pl
from jax.experimental.pallas import tpu as pltpu
from jax.experimental.pallas import tpu_sc as plsc

N = 10000
E = 320000
D_IN = 128
H1 = 128
H2 = 64
NC_OUT = 10

NCORES = 2
NSUB = 16
NWORK = NCORES * NSUB
CHUNK = 128
CHUNKS_PW = 80
IB = 32
CF_FAST = 128
CF_SLOW = 32
NCHUNKS = NWORK * CHUNKS_PW // 2 * 2
FAST_CORE = 0
EPW = CHUNK * CHUNKS_PW
EPAD = EPW * NWORK
NPAD = 10240
ROWS_PT = NPAD // NSUB
RB_CHUNKS = ROWS_PT // CHUNK

_mesh = plsc.VectorSubcoreMesh(core_axis_name="c", subcore_axis_name="s")


@functools.partial(
    pl.kernel,
    out_type=jax.ShapeDtypeStruct((NCORES, NPAD), jnp.float32),
    mesh=_mesh,
    scratch_types=[
        pltpu.VMEM_SHARED((NPAD,), jnp.float32),
        pltpu.VMEM((CHUNKS_PW, CHUNK), jnp.int32),
        pltpu.VMEM((CHUNK,), jnp.float32),
        pltpu.VMEM((ROWS_PT,), jnp.float32),
    ],
)
def _deg_kernel(dst_hbm, deg_out, acc, dst_v, ones_v, buf_v):
    cc = lax.axis_index("c")
    ss = lax.axis_index("s")
    w = cc * NSUB + ss

    pltpu.sync_copy(dst_hbm.at[pl.ds(CHUNKS_PW * w, CHUNKS_PW)], dst_v)
    for i in range(ROWS_PT // 16):
        buf_v[pl.ds(i * 16, 16)] = jnp.zeros((16,), jnp.float32)
    for i in range(CHUNK // 16):
        ones_v[pl.ds(i * 16, 16)] = jnp.ones((16,), jnp.float32)
    pltpu.sync_copy(buf_v, acc.at[pl.ds(ROWS_PT * ss, ROWS_PT)])
    plsc.subcore_barrier()

    @pl.loop(0, CHUNKS_PW)
    def _(i):
        pltpu.sync_copy(ones_v, acc.at[dst_v.at[i]], add=True)

    plsc.subcore_barrier()
    pltpu.sync_copy(acc.at[pl.ds(ROWS_PT * ss, ROWS_PT)], buf_v)
    pltpu.sync_copy(buf_v, deg_out.at[cc, pl.ds(ROWS_PT * ss, ROWS_PT)])


def _make_agg_kernel(width):
    @functools.partial(
        pl.kernel,
        out_type=jax.ShapeDtypeStruct((NCORES, NPAD, width), jnp.float32),
        mesh=_mesh,
        scratch_types=[
            pltpu.VMEM_SHARED((NPAD, width), jnp.float32),
            pltpu.VMEM((IB, CHUNK), jnp.int32),
            pltpu.VMEM((IB, CHUNK), jnp.int32),
            pltpu.VMEM((CHUNK, width), jnp.float32),
            pltpu.VMEM((CHUNK, width), jnp.float32),
            pltpu.SemaphoreType.DMA,
            pltpu.SemaphoreType.DMA,
        ],
    )
    def _agg(g_hbm, src_hbm, dst_hbm, zz_hbm, out, acc,
             src_v, dst_v, rows0, rows1, sem0, sem1):
        cc = lax.axis_index("c")
        ss = lax.axis_index("s")

        pltpu.sync_copy(zz_hbm, rows0)
        for k in range(RB_CHUNKS):
            pltpu.sync_copy(rows0, acc.at[pl.ds(ROWS_PT * ss + CHUNK * k, CHUNK)])
        plsc.subcore_barrier()

        def run(start_chunk, nblk):
            for blk in range(nblk):
                base = start_chunk + IB * blk
                pltpu.sync_copy(src_hbm.at[pl.ds(base, IB)], src_v)
                pltpu.sync_copy(dst_hbm.at[pl.ds(base, IB)], dst_v)
                pltpu.async_copy(g_hbm.at[src_v.at[0]], rows0, sem0)
                pltpu.async_copy(g_hbm.at[src_v.at[1]], rows1, sem1)

                @pl.loop(0, IB // 2)
                def _(gi):
                    c0 = 2 * gi
                    c1 = 2 * gi + 1
                    pltpu.make_async_copy(g_hbm.at[src_v.at[c0]], rows0, sem0).wait()
                    pltpu.sync_copy(rows0, acc.at[dst_v.at[c0]], add=True)

                    @pl.when(c0 + 2 < IB)
                    def _():
                        pltpu.async_copy(g_hbm.at[src_v.at[c0 + 2]], rows0, sem0)

                    pltpu.make_async_copy(g_hbm.at[src_v.at[c1]], rows1, sem1).wait()
                    pltpu.sync_copy(rows1, acc.at[dst_v.at[c1]], add=True)

                    @pl.when(c1 + 2 < IB)
                    def _():
                        pltpu.async_copy(g_hbm.at[src_v.at[c1 + 2]], rows1, sem1)

        @pl.when(cc == FAST_CORE)
        def _():
            run(ss * CF_FAST, CF_FAST // IB)

        @pl.when(cc != FAST_CORE)
        def _():
            run(NSUB * CF_FAST + ss * CF_SLOW, CF_SLOW // IB)

        plsc.subcore_barrier()
        for k in range(RB_CHUNKS):
            base = ROWS_PT * ss + CHUNK * k
            pltpu.sync_copy(acc.at[pl.ds(base, CHUNK)], rows0)
            pltpu.sync_copy(rows0, out.at[cc, pl.ds(base, CHUNK)])

    return _agg


_agg128 = _make_agg_kernel(H1)
H2P = 128


_BLK = 1000
_GRID = N // _BLK


def _tc1_body(p0, p1, x, w1, g1_out, dinv_out):
    deg = p0[...] + p1[...] + 1.0
    dinv = lax.rsqrt(deg)
    h = jnp.dot(x[...], w1[...], preferred_element_type=jnp.float32)
    g1_out[...] = h * dinv
    dinv_out[...] = dinv


def _tc2_body(p, g1, dinv, b1, w2, g2_out):
    pv = p[...]
    s = (pv[0] + pv[1] + g1[...]) * dinv[...]
    h1 = jnp.maximum(s + b1[...], 0.0)
    h2 = jnp.dot(h1, w2[...], preferred_element_type=jnp.float32)
    g2_out[...] = h2 * dinv[...]


def _tc3_body(q, g2, dinv, b2, wc, bc, out):
    qv = q[...]
    s = (qv[0] + qv[1] + g2[...]) * dinv[...]
    h2 = jnp.maximum(s + b2[...], 0.0)
    out[...] = jnp.dot(h2, wc[...], preferred_element_type=jnp.float32) + bc[...]


def _row_spec(width):
    return pl.BlockSpec((_BLK, width), lambda i: (i, 0))


def _pair_spec(width):
    return pl.BlockSpec((NCORES, _BLK, width), lambda i: (0, i, 0))


def _full_spec(a, b):
    return pl.BlockSpec((a, b), lambda i: (0, 0))


_tc1 = pl.pallas_call(
    _tc1_body,
    grid=(_GRID,),
    in_specs=[_row_spec(1), _row_spec(1), _row_spec(D_IN), _full_spec(D_IN, H1)],
    out_specs=[_row_spec(H1), _row_spec(1)],
    out_shape=[
        jax.ShapeDtypeStruct((N, H1), jnp.float32),
        jax.ShapeDtypeStruct((N, 1), jnp.float32),
    ],
)

_tc2 = pl.pallas_call(
    _tc2_body,
    grid=(_GRID,),
    in_specs=[_pair_spec(H1), _row_spec(H1), _row_spec(1),
              _full_spec(1, H1), _full_spec(H1, H2P)],
    out_specs=[_row_spec(H2P)],
    out_shape=[jax.ShapeDtypeStruct((N, H2P), jnp.float32)],
)

_tc3 = pl.pallas_call(
    _tc3_body,
    grid=(_GRID,),
    in_specs=[_pair_spec(H2P), _row_spec(H2P), _row_spec(1),
              _full_spec(1, H2P), _full_spec(H2P, NC_OUT), _full_spec(1, NC_OUT)],
    out_specs=[_row_spec(NC_OUT)],
    out_shape=[jax.ShapeDtypeStruct((N, NC_OUT), jnp.float32)],
)


def kernel(x, edge_index, W1, b1, W2, b2, Wc, bc):
    pad = EPAD - E
    src = jnp.concatenate([edge_index[0], jnp.zeros((pad,), jnp.int32)])
    dst = jnp.concatenate([edge_index[1], jnp.full((pad,), NPAD - 1, jnp.int32)])
    src_r = src.reshape(NCHUNKS, CHUNK)
    dst_r = dst.reshape(NCHUNKS, CHUNK)
    zz = jnp.zeros((CHUNK, H1), jnp.float32)

    degp = _deg_kernel(dst_r)
    p0 = degp[0, :N].reshape(N, 1)
    p1 = degp[1, :N].reshape(N, 1)

    W2p = jnp.pad(W2, ((0, 0), (0, H2P - H2)))
    b2p = jnp.pad(b2, (0, H2P - H2)).reshape(1, H2P)
    Wcp = jnp.pad(Wc, ((0, H2P - H2), (0, 0)))

    g1, dinv = _tc1(p0, p1, x, W1)
    part1 = _agg128(g1, src_r, dst_r, zz)
    (g2,) = _tc2(part1, g1, dinv, b1.reshape(1, H1), W2p)
    part2 = _agg128(g2, src_r, dst_r, zz)
    (out,) = _tc3(part2, g2, dinv, b2p, Wcp, bc.reshape(1, NC_OUT))
    return out


# speedup vs baseline: 31.4024x; 3.0269x over previous
import functools

import jax
import jax.numpy as jnp
from jax import lax
from jax.experimental import pallas as pl
from jax.experimental.pallas import tpu as pltpu
from jax.experimental.pallas import tpu_sc as plsc

N = 10000
E = 320000
D_IN = 128
H1 = 128
H2 = 64
NC_OUT = 10

NCORES = 2
NSUB = 16
NWORK = NCORES * NSUB
CHUNK = 128
CHUNKS_PW = 80
IB = 40
CF_FAST = 80
CF_SLOW = 80
NCHUNKS = NWORK * CHUNKS_PW // 2 * 2
FAST_CORE = 0
EPW = CHUNK * CHUNKS_PW
EPAD = EPW * NWORK
NPAD = 10240
ROWS_PT = NPAD // NSUB
RB_CHUNKS = ROWS_PT // CHUNK

_mesh = plsc.VectorSubcoreMesh(core_axis_name="c", subcore_axis_name="s")


@functools.partial(
    pl.kernel,
    out_type=jax.ShapeDtypeStruct((NCORES, NPAD), jnp.float32),
    mesh=_mesh,
    scratch_types=[
        pltpu.VMEM_SHARED((NPAD,), jnp.float32),
        pltpu.VMEM((CHUNKS_PW, CHUNK), jnp.int32),
        pltpu.VMEM((CHUNK,), jnp.float32),
        pltpu.VMEM((ROWS_PT,), jnp.float32),
    ],
)
def _deg_kernel(dst_hbm, deg_out, acc, dst_v, ones_v, buf_v):
    cc = lax.axis_index("c")
    ss = lax.axis_index("s")
    w = cc * NSUB + ss

    pltpu.sync_copy(dst_hbm.at[pl.ds(CHUNKS_PW * w, CHUNKS_PW)], dst_v)
    for i in range(ROWS_PT // 16):
        buf_v[pl.ds(i * 16, 16)] = jnp.zeros((16,), jnp.float32)
    for i in range(CHUNK // 16):
        ones_v[pl.ds(i * 16, 16)] = jnp.ones((16,), jnp.float32)
    pltpu.sync_copy(buf_v, acc.at[pl.ds(ROWS_PT * ss, ROWS_PT)])
    plsc.subcore_barrier()

    @pl.loop(0, CHUNKS_PW)
    def _(i):
        pltpu.sync_copy(ones_v, acc.at[dst_v.at[i]], add=True)

    plsc.subcore_barrier()
    pltpu.sync_copy(acc.at[pl.ds(ROWS_PT * ss, ROWS_PT)], buf_v)
    pltpu.sync_copy(buf_v, deg_out.at[cc, pl.ds(ROWS_PT * ss, ROWS_PT)])


def _make_agg_kernel(width):
    @functools.partial(
        pl.kernel,
        out_type=jax.ShapeDtypeStruct((NCORES, NPAD, width), jnp.float32),
        mesh=_mesh,
        scratch_types=[
            pltpu.VMEM_SHARED((NPAD, width), jnp.float32),
            pltpu.VMEM((IB, CHUNK), jnp.int32),
            pltpu.VMEM((IB, CHUNK), jnp.int32),
            pltpu.VMEM((CHUNK, width), jnp.float32),
            pltpu.VMEM((CHUNK, width), jnp.float32),
            pltpu.SemaphoreType.DMA,
            pltpu.SemaphoreType.DMA,
        ],
    )
    def _agg(g_hbm, src_hbm, dst_hbm, zz_hbm, out, acc,
             src_v, dst_v, rows0, rows1, sem0, sem1):
        cc = lax.axis_index("c")
        ss = lax.axis_index("s")

        pltpu.sync_copy(zz_hbm, rows0)
        for k in range(RB_CHUNKS):
            pltpu.sync_copy(rows0, acc.at[pl.ds(ROWS_PT * ss + CHUNK * k, CHUNK)])
        plsc.subcore_barrier()

        def run(start_chunk, nblk):
            for blk in range(nblk):
                base = start_chunk + IB * blk
                pltpu.sync_copy(src_hbm.at[pl.ds(base, IB)], src_v)
                pltpu.sync_copy(dst_hbm.at[pl.ds(base, IB)], dst_v)
                pltpu.async_copy(g_hbm.at[src_v.at[0]], rows0, sem0)
                pltpu.async_copy(g_hbm.at[src_v.at[1]], rows1, sem1)

                @pl.loop(0, IB // 2)
                def _(gi):
                    c0 = 2 * gi
                    c1 = 2 * gi + 1
                    pltpu.make_async_copy(g_hbm.at[src_v.at[c0]], rows0, sem0).wait()
                    pltpu.sync_copy(rows0, acc.at[dst_v.at[c0]], add=True)

                    @pl.when(c0 + 2 < IB)
                    def _():
                        pltpu.async_copy(g_hbm.at[src_v.at[c0 + 2]], rows0, sem0)

                    pltpu.make_async_copy(g_hbm.at[src_v.at[c1]], rows1, sem1).wait()
                    pltpu.sync_copy(rows1, acc.at[dst_v.at[c1]], add=True)

                    @pl.when(c1 + 2 < IB)
                    def _():
                        pltpu.async_copy(g_hbm.at[src_v.at[c1 + 2]], rows1, sem1)

        @pl.when(cc == FAST_CORE)
        def _():
            run(ss * CF_FAST, CF_FAST // IB)

        @pl.when(cc != FAST_CORE)
        def _():
            run(NSUB * CF_FAST + ss * CF_SLOW, CF_SLOW // IB)

        plsc.subcore_barrier()
        for k in range(RB_CHUNKS):
            base = ROWS_PT * ss + CHUNK * k
            pltpu.sync_copy(acc.at[pl.ds(base, CHUNK)], rows0)
            pltpu.sync_copy(rows0, out.at[cc, pl.ds(base, CHUNK)])

    return _agg


_agg128 = _make_agg_kernel(H1)
H2P = 128


_BLK = 1000
_GRID = N // _BLK


def _tc1_body(p0, p1, x, w1, g1_out, dinv_out):
    deg = p0[...] + p1[...] + 1.0
    dinv = lax.rsqrt(deg)
    h = jnp.dot(x[...], w1[...], preferred_element_type=jnp.float32)
    g1_out[...] = h * dinv
    dinv_out[...] = dinv


def _tc2_body(p, g1, dinv, b1, w2, g2_out):
    pv = p[...]
    s = (pv[0] + pv[1] + g1[...]) * dinv[...]
    h1 = jnp.maximum(s + b1[...], 0.0)
    h2 = jnp.dot(h1, w2[...], preferred_element_type=jnp.float32)
    g2_out[...] = h2 * dinv[...]


def _tc3_body(q, g2, dinv, b2, wc, bc, out):
    qv = q[...]
    s = (qv[0] + qv[1] + g2[...]) * dinv[...]
    h2 = jnp.maximum(s + b2[...], 0.0)
    out[...] = jnp.dot(h2, wc[...], preferred_element_type=jnp.float32) + bc[...]


def _row_spec(width):
    return pl.BlockSpec((_BLK, width), lambda i: (i, 0))


def _pair_spec(width):
    return pl.BlockSpec((NCORES, _BLK, width), lambda i: (0, i, 0))


def _full_spec(a, b):
    return pl.BlockSpec((a, b), lambda i: (0, 0))


_tc1 = pl.pallas_call(
    _tc1_body,
    grid=(_GRID,),
    in_specs=[_row_spec(1), _row_spec(1), _row_spec(D_IN), _full_spec(D_IN, H1)],
    out_specs=[_row_spec(H1), _row_spec(1)],
    out_shape=[
        jax.ShapeDtypeStruct((N, H1), jnp.float32),
        jax.ShapeDtypeStruct((N, 1), jnp.float32),
    ],
)

_tc2 = pl.pallas_call(
    _tc2_body,
    grid=(_GRID,),
    in_specs=[_pair_spec(H1), _row_spec(H1), _row_spec(1),
              _full_spec(1, H1), _full_spec(H1, H2P)],
    out_specs=[_row_spec(H2P)],
    out_shape=[jax.ShapeDtypeStruct((N, H2P), jnp.float32)],
)

_tc3 = pl.pallas_call(
    _tc3_body,
    grid=(_GRID,),
    in_specs=[_pair_spec(H2P), _row_spec(H2P), _row_spec(1),
              _full_spec(1, H2P), _full_spec(H2P, NC_OUT), _full_spec(1, NC_OUT)],
    out_specs=[_row_spec(NC_OUT)],
    out_shape=[jax.ShapeDtypeStruct((N, NC_OUT), jnp.float32)],
)


def kernel(x, edge_index, W1, b1, W2, b2, Wc, bc):
    pad = EPAD - E
    pr = jnp.arange(pad, dtype=jnp.int32)
    src = jnp.concatenate([edge_index[0], pr % N])
    dst = jnp.concatenate([edge_index[1], N + pr % (NPAD - N)])
    src_r = src.reshape(NCHUNKS, CHUNK)
    dst_r = dst.reshape(NCHUNKS, CHUNK)
    zz = jnp.zeros((CHUNK, H1), jnp.float32)

    degp = _deg_kernel(dst_r)
    p0 = degp[0, :N].reshape(N, 1)
    p1 = degp[1, :N].reshape(N, 1)

    W2p = jnp.pad(W2, ((0, 0), (0, H2P - H2)))
    b2p = jnp.pad(b2, (0, H2P - H2)).reshape(1, H2P)
    Wcp = jnp.pad(Wc, ((0, H2P - H2), (0, 0)))

    g1, dinv = _tc1(p0, p1, x, W1)
    part1 = _agg128(g1, src_r, dst_r, zz)
    (g2,) = _tc2(part1, g1, dinv, b1.reshape(1, H1), W2p)
    part2 = _agg128(g2, src_r, dst_r, zz)
    (out,) = _tc3(part2, g2, dinv, b2p, Wcp, bc.reshape(1, NC_OUT))
    return out


# speedup vs baseline: 31.4136x; 1.0004x over previous
import functools

import numpy as np
import jax
import jax.numpy as jnp
from jax import lax
from jax.experimental import pallas as pl
from jax.experimental.pallas import tpu as pltpu
from jax.experimental.pallas import tpu_sc as plsc

N = 10000
E = 320000
D_IN = 128
H1 = 128
H2 = 64
NC_OUT = 10

NCORES = 2
NSUB = 16
NWORK = NCORES * NSUB
CHUNK = 128
CHUNKS_PW = 80
IB = 40
CF_FAST = 80
CF_SLOW = 80
NCHUNKS = NWORK * CHUNKS_PW // 2 * 2
FAST_CORE = 0
EPW = CHUNK * CHUNKS_PW
EPAD = EPW * NWORK
NPAD = 10240
ROWS_PT = NPAD // NSUB
RB_CHUNKS = ROWS_PT // CHUNK

_mesh = plsc.VectorSubcoreMesh(core_axis_name="c", subcore_axis_name="s")

_PAD_SRC = (np.arange(EPAD - E) % N).astype(np.int32)
_PAD_DST = (N + np.arange(EPAD - E) % (NPAD - N)).astype(np.int32)


@functools.partial(
    pl.kernel,
    out_type=jax.ShapeDtypeStruct((NCORES, NPAD), jnp.float32),
    mesh=_mesh,
    scratch_types=[
        pltpu.VMEM_SHARED((NPAD,), jnp.float32),
        pltpu.VMEM((CHUNKS_PW, CHUNK), jnp.int32),
        pltpu.VMEM((CHUNK,), jnp.float32),
        pltpu.VMEM((ROWS_PT,), jnp.float32),
    ],
)
def _deg_kernel(dst_hbm, deg_out, acc, dst_v, ones_v, buf_v):
    cc = lax.axis_index("c")
    ss = lax.axis_index("s")
    w = cc * NSUB + ss

    pltpu.sync_copy(dst_hbm.at[pl.ds(CHUNKS_PW * w, CHUNKS_PW)], dst_v)
    for i in range(ROWS_PT // 16):
        buf_v[pl.ds(i * 16, 16)] = jnp.zeros((16,), jnp.float32)
    for i in range(CHUNK // 16):
        ones_v[pl.ds(i * 16, 16)] = jnp.ones((16,), jnp.float32)
    pltpu.sync_copy(buf_v, acc.at[pl.ds(ROWS_PT * ss, ROWS_PT)])
    plsc.subcore_barrier()

    @pl.loop(0, CHUNKS_PW)
    def _(i):
        pltpu.sync_copy(ones_v, acc.at[dst_v.at[i]], add=True)

    plsc.subcore_barrier()
    pltpu.sync_copy(acc.at[pl.ds(ROWS_PT * ss, ROWS_PT)], buf_v)
    pltpu.sync_copy(buf_v, deg_out.at[cc, pl.ds(ROWS_PT * ss, ROWS_PT)])


def _make_agg_kernel(width):
    @functools.partial(
        pl.kernel,
        out_type=jax.ShapeDtypeStruct((NCORES, NPAD, width), jnp.float32),
        mesh=_mesh,
        scratch_types=[
            pltpu.VMEM_SHARED((NPAD, width), jnp.float32),
            pltpu.VMEM((IB, CHUNK), jnp.int32),
            pltpu.VMEM((IB, CHUNK), jnp.int32),
            pltpu.VMEM((CHUNK, width), jnp.float32),
            pltpu.VMEM((CHUNK, width), jnp.float32),
            pltpu.SemaphoreType.DMA,
            pltpu.SemaphoreType.DMA,
        ],
    )
    def _agg(g_hbm, src_hbm, dst_hbm, zz_hbm, out, acc,
             src_v, dst_v, rows0, rows1, sem0, sem1):
        cc = lax.axis_index("c")
        ss = lax.axis_index("s")

        pltpu.sync_copy(zz_hbm, rows0)
        for k in range(RB_CHUNKS):
            pltpu.sync_copy(rows0, acc.at[pl.ds(ROWS_PT * ss + CHUNK * k, CHUNK)])
        plsc.subcore_barrier()

        def run(start_chunk, nblk):
            for blk in range(nblk):
                base = start_chunk + IB * blk
                pltpu.sync_copy(src_hbm.at[pl.ds(base, IB)], src_v)
                pltpu.sync_copy(dst_hbm.at[pl.ds(base, IB)], dst_v)
                pltpu.async_copy(g_hbm.at[src_v.at[0]], rows0, sem0)
                pltpu.async_copy(g_hbm.at[src_v.at[1]], rows1, sem1)

                @pl.loop(0, IB // 2)
                def _(gi):
                    c0 = 2 * gi
                    c1 = 2 * gi + 1
                    pltpu.make_async_copy(g_hbm.at[src_v.at[c0]], rows0, sem0).wait()
                    pltpu.sync_copy(rows0, acc.at[dst_v.at[c0]], add=True)

                    @pl.when(c0 + 2 < IB)
                    def _():
                        pltpu.async_copy(g_hbm.at[src_v.at[c0 + 2]], rows0, sem0)

                    pltpu.make_async_copy(g_hbm.at[src_v.at[c1]], rows1, sem1).wait()
                    pltpu.sync_copy(rows1, acc.at[dst_v.at[c1]], add=True)

                    @pl.when(c1 + 2 < IB)
                    def _():
                        pltpu.async_copy(g_hbm.at[src_v.at[c1 + 2]], rows1, sem1)

        @pl.when(cc == FAST_CORE)
        def _():
            run(ss * CF_FAST, CF_FAST // IB)

        @pl.when(cc != FAST_CORE)
        def _():
            run(NSUB * CF_FAST + ss * CF_SLOW, CF_SLOW // IB)

        plsc.subcore_barrier()
        for k in range(RB_CHUNKS):
            base = ROWS_PT * ss + CHUNK * k
            pltpu.async_copy(acc.at[pl.ds(base, CHUNK)],
                             out.at[cc, pl.ds(base, CHUNK)], sem0)
        for k in range(RB_CHUNKS):
            base = ROWS_PT * ss + CHUNK * k
            pltpu.make_async_copy(acc.at[pl.ds(base, CHUNK)],
                                  out.at[cc, pl.ds(base, CHUNK)], sem0).wait()

    return _agg


_agg128 = _make_agg_kernel(H1)
H2P = 128


_BLK = 1000
_GRID = N // _BLK


def _tc1_body(p0, p1, x, w1, g1_out, dinv_out):
    deg = p0[...] + p1[...] + 1.0
    dinv = lax.rsqrt(deg)
    h = jnp.dot(x[...], w1[...], preferred_element_type=jnp.float32)
    g1_out[...] = h * dinv
    dinv_out[...] = dinv


def _tc2_body(p, g1, dinv, b1, w2, g2_out):
    pv = p[...]
    s = (pv[0] + pv[1] + g1[...]) * dinv[...]
    h1 = jnp.maximum(s + b1[...], 0.0)
    h2 = jnp.dot(h1, w2[...], preferred_element_type=jnp.float32)
    g2_out[...] = h2 * dinv[...]


def _tc3_body(q, g2, dinv, b2, wc, bc, out):
    qv = q[...]
    s = (qv[0] + qv[1] + g2[...]) * dinv[...]
    h2 = jnp.maximum(s + b2[...], 0.0)
    out[...] = jnp.dot(h2, wc[...], preferred_element_type=jnp.float32) + bc[...]


def _row_spec(width):
    return pl.BlockSpec((_BLK, width), lambda i: (i, 0))


def _pair_spec(width):
    return pl.BlockSpec((NCORES, _BLK, width), lambda i: (0, i, 0))


def _full_spec(a, b):
    return pl.BlockSpec((a, b), lambda i: (0, 0))


_tc1 = pl.pallas_call(
    _tc1_body,
    grid=(_GRID,),
    in_specs=[_row_spec(1), _row_spec(1), _row_spec(D_IN), _full_spec(D_IN, H1)],
    out_specs=[_row_spec(H1), _row_spec(1)],
    out_shape=[
        jax.ShapeDtypeStruct((N, H1), jnp.float32),
        jax.ShapeDtypeStruct((N, 1), jnp.float32),
    ],
)

_tc2 = pl.pallas_call(
    _tc2_body,
    grid=(_GRID,),
    in_specs=[_pair_spec(H1), _row_spec(H1), _row_spec(1),
              _full_spec(1, H1), _full_spec(H1, H2P)],
    out_specs=[_row_spec(H2P)],
    out_shape=[jax.ShapeDtypeStruct((N, H2P), jnp.float32)],
)

_tc3 = pl.pallas_call(
    _tc3_body,
    grid=(_GRID,),
    in_specs=[_pair_spec(H2P), _row_spec(H2P), _row_spec(1),
              _full_spec(1, H2P), _full_spec(H2P, NC_OUT), _full_spec(1, NC_OUT)],
    out_specs=[_row_spec(NC_OUT)],
    out_shape=[jax.ShapeDtypeStruct((N, NC_OUT), jnp.float32)],
)


def kernel(x, edge_index, W1, b1, W2, b2, Wc, bc):
    src = jnp.concatenate([edge_index[0], jnp.asarray(_PAD_SRC)])
    dst = jnp.concatenate([edge_index[1], jnp.asarray(_PAD_DST)])
    src_r = src.reshape(NCHUNKS, CHUNK)
    dst_r = dst.reshape(NCHUNKS, CHUNK)
    zz = jnp.zeros((CHUNK, H1), jnp.float32)

    degp = _deg_kernel(dst_r)
    p0 = degp[0, :N].reshape(N, 1)
    p1 = degp[1, :N].reshape(N, 1)

    W2p = jnp.pad(W2, ((0, 0), (0, H2P - H2)))
    b2p = jnp.pad(b2, (0, H2P - H2)).reshape(1, H2P)
    Wcp = jnp.pad(Wc, ((0, H2P - H2), (0, 0)))

    g1, dinv = _tc1(p0, p1, x, W1)
    part1 = _agg128(g1, src_r, dst_r, zz)
    (g2,) = _tc2(part1, g1, dinv, b1.reshape(1, H1), W2p)
    part2 = _agg128(g2, src_r, dst_r, zz)
    (out,) = _tc3(part2, g2, dinv, b2p, Wcp, bc.reshape(1, NC_OUT))
    return out
